# baseline (speedup 1.0000x reference)
import functools

import jax
import jax.numpy as jnp
from jax import lax
from jax.experimental import pallas as pl
from jax.experimental.pallas import tpu as pltpu
from jax.experimental.pallas import tpu_sc as plsc

VOCAB = 1000000
DIM = 64
WDIM = 128
BATCH = 16384

NUM_CORES = 2
NUM_SUBCORES = 16
NW = NUM_CORES * NUM_SUBCORES
BASE_COLS = 31232
EXTRA = 4
TAIL_START = 999936
TAIL_COLS = VOCAB - TAIL_START
CCOLS = 512
NPAIR = 31
WINDOW = 64
NIDX_G = BATCH // 16
OUT_ROWS = BATCH + NW
PACK_SHIFT = 15
COL_MASK = (1 << PACK_SHIFT) - 1

_mesh = plsc.VectorSubcoreMesh(core_axis_name="c", subcore_axis_name="s")


@functools.partial(
    pl.kernel,
    out_type=jax.ShapeDtypeStruct((OUT_ROWS, WDIM), jnp.float32),
    mesh=_mesh,
    scratch_types=[
        pltpu.VMEM((BATCH,), jnp.int32),
        pltpu.VMEM((BATCH + 16,), jnp.int32),
        pltpu.VMEM((DIM, CCOLS), jnp.float32),
        pltpu.VMEM((DIM, CCOLS), jnp.float32),
        pltpu.VMEM((WINDOW + 16,), jnp.int32),
        pltpu.VMEM((1, WINDOW), jnp.int32),
        pltpu.VMEM((WINDOW + 16, WDIM), jnp.float32),
        pltpu.SemaphoreType.DMA,
        pltpu.SemaphoreType.DMA,
        pltpu.SemaphoreType.DMA,
    ],
    compiler_params=pltpu.CompilerParams(
        use_tc_tiling_on_sc=True, needs_layout_passes=False),
)
def _sc_scan(idx_hbm, table_hbm, tail_hbm, out_hbm, idx_v, mp_v, ck_a, ck_b,
             wl_v, pos_w, outbuf, sem_a, sem_b, sem_s):
    wid = lax.axis_index("s") * NUM_CORES + lax.axis_index("c")
    lo = wid * BASE_COLS + jnp.minimum(wid, EXTRA) * 128
    n_w = BASE_COLS + jnp.where(wid < EXTRA, 128, 0)
    is_tail_w = wid == NW - 1
    hi = jnp.where(is_tail_w, VOCAB, lo + n_w)
    last_off = n_w - CCOLS
    sink = BATCH + wid
    iota16 = lax.iota(jnp.int32, 16)
    sink16 = jnp.full((16,), sink, jnp.int32)
    pad16 = jnp.full((16,), COL_MASK, jnp.int32) | (sink16 << PACK_SHIFT)
    wlpad16 = sink16 << PACK_SHIFT

    def off_k(k):
        return jnp.minimum(k * CCOLS, last_off)

    def wait_chunk(ck, sem):
        pltpu.make_async_copy(table_hbm.at[:, pl.ds(lo, CCOLS)], ck,
                              sem).wait()

    pltpu.async_copy(table_hbm.at[:, pl.ds(lo + off_k(0), CCOLS)], ck_a, sem_a)
    pltpu.async_copy(table_hbm.at[:, pl.ds(lo + off_k(1), CCOLS)], ck_b, sem_b)
    pltpu.sync_copy(idx_hbm, idx_v)

    for i in range((WINDOW + 16) // 16):
        wl_v[pl.ds(i * 16, 16)] = wlpad16
    for i in range(WINDOW // 16):
        pos_w[0, pl.ds(i * 16, 16)] = sink16
    pltpu.async_copy(outbuf.at[pl.ds(0, WINDOW)], out_hbm.at[pos_w.at[0]],
                     sem_s)

    def _route(i, cnt):
        for j in range(2):
            g = i * 2 + j
            v = idx_v[pl.ds(g * 16, 16)]
            m = (v >= lo) & (v < hi)
            nm = plsc.all_reduce_population_count(m)[0]

            @pl.when(nm > 0)
            def _emit(v=v, m=m, g=g, base=cnt):
                packed = (v - lo) | ((g * 16 + iota16) << PACK_SHIFT)
                pre = base + plsc.cumsum(m.astype(jnp.int32))
                plsc.store_scatter(mp_v, [pre - 1], packed, mask=m)

            cnt = cnt + nm
        return cnt

    cnt = lax.fori_loop(0, NIDX_G // 2, _route, jnp.int32(0))
    mp_v[pl.ds(cnt, 16)] = pad16
    ng = (cnt + 15) >> 4

    def _window(gather_fn, off, cw, base):
        def _rescan(g, wc):
            e = mp_v[pl.ds(g * 16, 16)]
            col = e & COL_MASK
            m2 = (col >= off) & (col < off + cw)
            pre = plsc.cumsum(m2.astype(jnp.int32)) + wc
            sel = m2 & (pre > base) & (pre <= base + WINDOW)
            plsc.store_scatter(wl_v, [pre - 1 - base], e - off, mask=sel)
            return pre[15]

        wc = lax.fori_loop(0, ng, _rescan, jnp.int32(0))
        wcn = jnp.clip(wc - base, 0, WINDOW)
        wl_v[pl.ds(wcn, 16)] = wlpad16

        pltpu.make_async_copy(out_hbm.at[pl.ds(0, WINDOW)],
                              outbuf.at[pl.ds(0, WINDOW)], sem_s).wait()

        for i in range(WINDOW // 16):
            ew = wl_v[pl.ds(i * 16, 16)]
            ok = (i * 16 + iota16) < wcn
            pos_w[0, pl.ds(i * 16, 16)] = jnp.where(
                ok, lax.shift_right_logical(ew, PACK_SHIFT), sink16)

        def _extract(e, _):
            ew = wl_v[pl.ds(e * 16, 16)]
            lc = ew & COL_MASK
            slot = e * 16 + iota16
            for d in range(DIM):
                vals = gather_fn(jnp.full((16,), d, jnp.int32), lc)
                plsc.store_scatter(
                    outbuf, [slot, jnp.full((16,), d, jnp.int32)], vals)
            return 0

        lax.fori_loop(0, (wcn + 15) >> 4, _extract, 0)

        pltpu.async_copy(outbuf.at[pl.ds(0, WINDOW)],
                         out_hbm.at[pos_w.at[0]], sem_s)
        return wc

    def _process(gather_fn, off, cw):
        wc = _window(gather_fn, off, cw, jnp.int32(0))
        nwin = (wc + (WINDOW - 1)) >> 6

        def _more(s, _):
            _window(gather_fn, off, cw, s * WINDOW)
            return 0

        lax.fori_loop(1, nwin, _more, 0)

    def _gather_pair(d16, lc):
        in_a = lc < CCOLS
        va = plsc.load_gather(ck_a, [d16, lc], mask=in_a)
        vb = plsc.load_gather(ck_b, [d16, lc - CCOLS],
                              mask=jnp.logical_not(in_a))
        return jnp.where(in_a, va, vb)

    def _pair(kk, _):
        off_p = jnp.minimum(kk * 2 * CCOLS, n_w - 2 * CCOLS)
        wait_chunk(ck_a, sem_a)
        wait_chunk(ck_b, sem_b)
        _process(_gather_pair, off_p, 2 * CCOLS)
        off_n = jnp.minimum((kk + 1) * 2 * CCOLS, n_w - 2 * CCOLS)
        pltpu.async_copy(table_hbm.at[:, pl.ds(lo + off_n, CCOLS)],
                         ck_a, sem_a)
        pltpu.async_copy(table_hbm.at[:, pl.ds(lo + off_n + CCOLS, CCOLS)],
                         ck_b, sem_b)
        return 0

    lax.fori_loop(0, NPAIR, _pair, 0)

    wait_chunk(ck_a, sem_a)
    wait_chunk(ck_b, sem_b)

    @pl.when(is_tail_w)
    def _tail():
        pltpu.sync_copy(tail_hbm, ck_a.at[:, pl.ds(0, 128)])
        _process(lambda d16, lc: plsc.load_gather(ck_a, [d16, lc]),
                 jnp.int32(BASE_COLS), TAIL_COLS)

    pltpu.make_async_copy(out_hbm.at[pl.ds(0, WINDOW)],
                          outbuf.at[pl.ds(0, WINDOW)], sem_s).wait()


_TC_ROWS = 2048


def _tc_narrow_body(wide_ref, out_ref):
    out_ref[...] = wide_ref[:, :DIM]


_tc_narrow = pl.pallas_call(
    _tc_narrow_body,
    grid=(BATCH // _TC_ROWS,),
    in_specs=[pl.BlockSpec((_TC_ROWS, WDIM), lambda i: (i, 0))],
    out_specs=pl.BlockSpec((_TC_ROWS, DIM), lambda i: (i, 0)),
    out_shape=jax.ShapeDtypeStruct((BATCH, DIM), jnp.float32),
)


def kernel(inputs, train_labels, embeddings):
    del train_labels
    table_t = embeddings.T
    tail_t = jnp.pad(embeddings[TAIL_START:], ((0, 128 - TAIL_COLS), (0, 0))).T
    wide = _sc_scan(inputs, table_t, tail_t)
    return _tc_narrow(wide)


# speedup vs baseline: 1.0577x; 1.0577x over previous
import functools

import jax
import jax.numpy as jnp
from jax import lax
from jax.experimental import pallas as pl
from jax.experimental.pallas import tpu as pltpu
from jax.experimental.pallas import tpu_sc as plsc

VOCAB = 1000000
DIM = 64
WDIM = 128
BATCH = 16384

NUM_CORES = 2
NUM_SUBCORES = 16
NW = NUM_CORES * NUM_SUBCORES
BASE_COLS = 31232
EXTRA = 4
TAIL_START = 999936
TAIL_COLS = VOCAB - TAIL_START
CCOLS = 512
NPAIR = 31
WINDOW = 64
NIDX_G = BATCH // 16
OUT_ROWS = BATCH + NW
PACK_SHIFT = 15
COL_MASK = (1 << PACK_SHIFT) - 1

_mesh = plsc.VectorSubcoreMesh(core_axis_name="c", subcore_axis_name="s")


@functools.partial(
    pl.kernel,
    out_type=jax.ShapeDtypeStruct((OUT_ROWS, WDIM), jnp.float32),
    mesh=_mesh,
    scratch_types=[
        pltpu.VMEM((BATCH,), jnp.int32),
        pltpu.VMEM((BATCH + 16,), jnp.int32),
        pltpu.VMEM((DIM, CCOLS), jnp.float32),
        pltpu.VMEM((DIM, CCOLS), jnp.float32),
        pltpu.VMEM((WINDOW + 16,), jnp.int32),
        pltpu.VMEM((1, WINDOW), jnp.int32),
        pltpu.VMEM((WINDOW + 16, WDIM), jnp.float32),
        pltpu.SemaphoreType.DMA,
        pltpu.SemaphoreType.DMA,
        pltpu.SemaphoreType.DMA,
    ],
    compiler_params=pltpu.CompilerParams(
        use_tc_tiling_on_sc=True, needs_layout_passes=False),
)
def _sc_scan(idx_hbm, table_hbm, tail_hbm, out_hbm, idx_v, mp_v, ck_a, ck_b,
             wl_v, pos_w, outbuf, sem_a, sem_b, sem_s):
    wid = lax.axis_index("s") * NUM_CORES + lax.axis_index("c")
    lo = wid * BASE_COLS + jnp.minimum(wid, EXTRA) * 128
    n_w = BASE_COLS + jnp.where(wid < EXTRA, 128, 0)
    is_tail_w = wid == NW - 1
    hi = jnp.where(is_tail_w, VOCAB, lo + n_w)
    last_off = n_w - CCOLS
    sink = BATCH + wid
    iota16 = lax.iota(jnp.int32, 16)
    sink16 = jnp.full((16,), sink, jnp.int32)
    pad16 = jnp.full((16,), COL_MASK, jnp.int32) | (sink16 << PACK_SHIFT)
    wlpad16 = sink16 << PACK_SHIFT

    def off_k(k):
        return jnp.minimum(k * CCOLS, last_off)

    def wait_chunk(ck, sem):
        pltpu.make_async_copy(table_hbm.at[:, pl.ds(lo, CCOLS)], ck,
                              sem).wait()

    pltpu.async_copy(table_hbm.at[:, pl.ds(lo + off_k(0), CCOLS)], ck_a, sem_a)
    pltpu.async_copy(table_hbm.at[:, pl.ds(lo + off_k(1), CCOLS)], ck_b, sem_b)
    pltpu.sync_copy(idx_hbm, idx_v)

    for i in range((WINDOW + 16) // 16):
        wl_v[pl.ds(i * 16, 16)] = wlpad16
    for i in range(WINDOW // 16):
        pos_w[0, pl.ds(i * 16, 16)] = sink16
    pltpu.async_copy(outbuf.at[pl.ds(0, WINDOW)], out_hbm.at[pos_w.at[0]],
                     sem_s)

    def _route(i, cnt):
        v = idx_v[pl.ds(i * 16, 16)]
        m = (v >= lo) & (v < hi)
        packed = (v - lo) | ((i * 16 + iota16) << PACK_SHIFT)
        pre = cnt + plsc.cumsum(m.astype(jnp.int32))
        plsc.store_scatter(mp_v, [pre - 1], packed, mask=m)
        return pre[15]

    cnt = lax.fori_loop(0, NIDX_G, _route, jnp.int32(0))
    mp_v[pl.ds(cnt, 16)] = pad16
    ng = (cnt + 15) >> 4

    def _window(gather_fn, off, cw, base):
        def _rescan(g, wc):
            e = mp_v[pl.ds(g * 16, 16)]
            col = e & COL_MASK
            m2 = (col >= off) & (col < off + cw)
            pre = plsc.cumsum(m2.astype(jnp.int32)) + wc
            sel = m2 & (pre > base) & (pre <= base + WINDOW)
            plsc.store_scatter(wl_v, [pre - 1 - base], e - off, mask=sel)
            return pre[15]

        wc = lax.fori_loop(0, ng, _rescan, jnp.int32(0))
        wcn = jnp.clip(wc - base, 0, WINDOW)
        wl_v[pl.ds(wcn, 16)] = wlpad16

        pltpu.make_async_copy(out_hbm.at[pl.ds(0, WINDOW)],
                              outbuf.at[pl.ds(0, WINDOW)], sem_s).wait()

        for i in range(WINDOW // 16):
            ew = wl_v[pl.ds(i * 16, 16)]
            ok = (i * 16 + iota16) < wcn
            pos_w[0, pl.ds(i * 16, 16)] = jnp.where(
                ok, lax.shift_right_logical(ew, PACK_SHIFT), sink16)

        def _extract(e, _):
            ew = wl_v[pl.ds(e * 16, 16)]
            lc = ew & COL_MASK
            slot = e * 16 + iota16
            for d in range(DIM):
                vals = gather_fn(jnp.full((16,), d, jnp.int32), lc)
                plsc.store_scatter(
                    outbuf, [slot, jnp.full((16,), d, jnp.int32)], vals)
            return 0

        lax.fori_loop(0, (wcn + 15) >> 4, _extract, 0)

        pltpu.async_copy(outbuf.at[pl.ds(0, WINDOW)],
                         out_hbm.at[pos_w.at[0]], sem_s)
        return wc

    def _process(gather_fn, off, cw):
        wc = _window(gather_fn, off, cw, jnp.int32(0))
        nwin = (wc + (WINDOW - 1)) >> 6

        def _more(s, _):
            _window(gather_fn, off, cw, s * WINDOW)
            return 0

        lax.fori_loop(1, nwin, _more, 0)

    def _gather_pair(d16, lc):
        in_a = lc < CCOLS
        va = plsc.load_gather(ck_a, [d16, lc], mask=in_a)
        vb = plsc.load_gather(ck_b, [d16, lc - CCOLS],
                              mask=jnp.logical_not(in_a))
        return jnp.where(in_a, va, vb)

    def _pair(kk, _):
        off_p = jnp.minimum(kk * 2 * CCOLS, n_w - 2 * CCOLS)
        wait_chunk(ck_a, sem_a)
        wait_chunk(ck_b, sem_b)
        _process(_gather_pair, off_p, 2 * CCOLS)
        off_n = jnp.minimum((kk + 1) * 2 * CCOLS, n_w - 2 * CCOLS)
        pltpu.async_copy(table_hbm.at[:, pl.ds(lo + off_n, CCOLS)],
                         ck_a, sem_a)
        pltpu.async_copy(table_hbm.at[:, pl.ds(lo + off_n + CCOLS, CCOLS)],
                         ck_b, sem_b)
        return 0

    lax.fori_loop(0, NPAIR, _pair, 0)

    wait_chunk(ck_a, sem_a)
    wait_chunk(ck_b, sem_b)

    @pl.when(is_tail_w)
    def _tail():
        pltpu.sync_copy(tail_hbm, ck_a.at[:, pl.ds(0, 128)])
        _process(lambda d16, lc: plsc.load_gather(ck_a, [d16, lc]),
                 jnp.int32(BASE_COLS), TAIL_COLS)

    pltpu.make_async_copy(out_hbm.at[pl.ds(0, WINDOW)],
                          outbuf.at[pl.ds(0, WINDOW)], sem_s).wait()


_TC_ROWS = 2048


def _tc_narrow_body(wide_ref, out_ref):
    out_ref[...] = wide_ref[:, :DIM].T


_tc_narrow = pl.pallas_call(
    _tc_narrow_body,
    grid=(BATCH // _TC_ROWS,),
    in_specs=[pl.BlockSpec((_TC_ROWS, WDIM), lambda i: (i, 0))],
    out_specs=pl.BlockSpec((DIM, _TC_ROWS), lambda i: (0, i)),
    out_shape=jax.ShapeDtypeStruct((DIM, BATCH), jnp.float32),
)


def kernel(inputs, train_labels, embeddings):
    del train_labels
    table_t = embeddings.T
    tail_t = jnp.pad(embeddings[TAIL_START:], ((0, 128 - TAIL_COLS), (0, 0))).T
    wide = _sc_scan(inputs, table_t, tail_t)
    return _tc_narrow(wide).T


# speedup vs baseline: 1.0619x; 1.0040x over previous
import functools

import jax
import jax.numpy as jnp
from jax import lax
from jax.experimental import pallas as pl
from jax.experimental.pallas import tpu as pltpu
from jax.experimental.pallas import tpu_sc as plsc

VOCAB = 1000000
DIM = 64
WDIM = 128
BATCH = 16384

NUM_CORES = 2
NUM_SUBCORES = 16
NW = NUM_CORES * NUM_SUBCORES
BASE_COLS = 31232
EXTRA = 4
TAIL_START = 999936
TAIL_COLS = VOCAB - TAIL_START
CCOLS = 512
NPAIR = 31
WINDOW = 64
NIDX_G = BATCH // 16
OUT_ROWS = BATCH + NW
PACK_SHIFT = 15
COL_MASK = (1 << PACK_SHIFT) - 1

_mesh = plsc.VectorSubcoreMesh(core_axis_name="c", subcore_axis_name="s")


@functools.partial(
    pl.kernel,
    out_type=jax.ShapeDtypeStruct((OUT_ROWS, WDIM), jnp.float32),
    mesh=_mesh,
    scratch_types=[
        pltpu.VMEM((BATCH,), jnp.int32),
        pltpu.VMEM((BATCH + 16,), jnp.int32),
        pltpu.VMEM((DIM, 2 * CCOLS), jnp.float32),
        pltpu.VMEM((WINDOW + 16,), jnp.int32),
        pltpu.VMEM((1, WINDOW), jnp.int32),
        pltpu.VMEM((WINDOW + 16, WDIM), jnp.float32),
        pltpu.SemaphoreType.DMA,
        pltpu.SemaphoreType.DMA,
        pltpu.SemaphoreType.DMA,
    ],
    compiler_params=pltpu.CompilerParams(
        use_tc_tiling_on_sc=True, needs_layout_passes=False),
)
def _sc_scan(idx_hbm, table_hbm, tail_hbm, out_hbm, idx_v, mp_v, ck_v,
             wl_v, pos_w, outbuf, sem_a, sem_b, sem_s):
    wid = lax.axis_index("s") * NUM_CORES + lax.axis_index("c")
    lo = wid * BASE_COLS + jnp.minimum(wid, EXTRA) * 128
    n_w = BASE_COLS + jnp.where(wid < EXTRA, 128, 0)
    is_tail_w = wid == NW - 1
    hi = jnp.where(is_tail_w, VOCAB, lo + n_w)
    last_off = n_w - CCOLS
    sink = BATCH + wid
    iota16 = lax.iota(jnp.int32, 16)
    sink16 = jnp.full((16,), sink, jnp.int32)
    pad16 = jnp.full((16,), COL_MASK, jnp.int32) | (sink16 << PACK_SHIFT)
    wlpad16 = sink16 << PACK_SHIFT

    def off_k(k):
        return jnp.minimum(k * CCOLS, last_off)

    def fire_pair(off_p, sa, sb):
        pltpu.async_copy(table_hbm.at[:, pl.ds(lo + off_p, CCOLS)],
                         ck_v.at[:, pl.ds(0, CCOLS)], sa)
        pltpu.async_copy(table_hbm.at[:, pl.ds(lo + off_p + CCOLS, CCOLS)],
                         ck_v.at[:, pl.ds(CCOLS, CCOLS)], sb)

    def wait_pair(sa, sb):
        pltpu.make_async_copy(table_hbm.at[:, pl.ds(lo, CCOLS)],
                              ck_v.at[:, pl.ds(0, CCOLS)], sa).wait()
        pltpu.make_async_copy(table_hbm.at[:, pl.ds(lo, CCOLS)],
                              ck_v.at[:, pl.ds(CCOLS, CCOLS)], sb).wait()

    fire_pair(jnp.int32(0), sem_a, sem_b)
    pltpu.sync_copy(idx_hbm, idx_v)

    for i in range((WINDOW + 16) // 16):
        wl_v[pl.ds(i * 16, 16)] = wlpad16
    for i in range(WINDOW // 16):
        pos_w[0, pl.ds(i * 16, 16)] = sink16
    pltpu.async_copy(outbuf.at[pl.ds(0, WINDOW)], out_hbm.at[pos_w.at[0]],
                     sem_s)

    def _route(i, cnt):
        v = idx_v[pl.ds(i * 16, 16)]
        m = (v >= lo) & (v < hi)
        packed = (v - lo) | ((i * 16 + iota16) << PACK_SHIFT)
        pre = cnt + plsc.cumsum(m.astype(jnp.int32))
        plsc.store_scatter(mp_v, [pre - 1], packed, mask=m)
        return pre[15]

    cnt = lax.fori_loop(0, NIDX_G, _route, jnp.int32(0))
    mp_v[pl.ds(cnt, 16)] = pad16
    ng = (cnt + 15) >> 4

    def _window(gather_fn, off, cw, base):
        def _rescan(g, wc):
            e = mp_v[pl.ds(g * 16, 16)]
            col = e & COL_MASK
            m2 = (col >= off) & (col < off + cw)
            pre = plsc.cumsum(m2.astype(jnp.int32)) + wc
            sel = m2 & (pre > base) & (pre <= base + WINDOW)
            plsc.store_scatter(wl_v, [pre - 1 - base], e - off, mask=sel)
            return pre[15]

        wc = lax.fori_loop(0, ng, _rescan, jnp.int32(0))
        wcn = jnp.clip(wc - base, 0, WINDOW)
        wl_v[pl.ds(wcn, 16)] = wlpad16

        pltpu.make_async_copy(out_hbm.at[pl.ds(0, WINDOW)],
                              outbuf.at[pl.ds(0, WINDOW)], sem_s).wait()

        for i in range(WINDOW // 16):
            ew = wl_v[pl.ds(i * 16, 16)]
            ok = (i * 16 + iota16) < wcn
            pos_w[0, pl.ds(i * 16, 16)] = jnp.where(
                ok, lax.shift_right_logical(ew, PACK_SHIFT), sink16)

        def _extract(e, _):
            ew = wl_v[pl.ds(e * 16, 16)]
            lc = ew & COL_MASK
            slot = e * 16 + iota16
            for d in range(DIM):
                vals = gather_fn(jnp.full((16,), d, jnp.int32), lc)
                plsc.store_scatter(
                    outbuf, [slot, jnp.full((16,), d, jnp.int32)], vals)
            return 0

        lax.fori_loop(0, (wcn + 15) >> 4, _extract, 0)

        pltpu.async_copy(outbuf.at[pl.ds(0, WINDOW)],
                         out_hbm.at[pos_w.at[0]], sem_s)
        return wc

    def _process(gather_fn, off, cw):
        wc = _window(gather_fn, off, cw, jnp.int32(0))
        nwin = (wc + (WINDOW - 1)) >> 6

        def _more(s, _):
            _window(gather_fn, off, cw, s * WINDOW)
            return 0

        lax.fori_loop(1, nwin, _more, 0)

    def _gather_pair(d16, lc):
        return plsc.load_gather(ck_v, [d16, lc])

    def _pair(kk, _):
        off_p = jnp.minimum(kk * 2 * CCOLS, n_w - 2 * CCOLS)
        wait_pair(sem_a, sem_b)
        _process(_gather_pair, off_p, 2 * CCOLS)
        off_n = jnp.minimum((kk + 1) * 2 * CCOLS, n_w - 2 * CCOLS)
        fire_pair(off_n, sem_a, sem_b)
        return 0

    lax.fori_loop(0, NPAIR, _pair, 0)

    wait_pair(sem_a, sem_b)

    @pl.when(is_tail_w)
    def _tail():
        pltpu.sync_copy(tail_hbm, ck_v.at[:, pl.ds(0, 128)])
        _process(_gather_pair, jnp.int32(BASE_COLS), TAIL_COLS)

    pltpu.make_async_copy(out_hbm.at[pl.ds(0, WINDOW)],
                          outbuf.at[pl.ds(0, WINDOW)], sem_s).wait()


_TC_ROWS = 2048


def _tc_narrow_body(wide_ref, out_ref):
    out_ref[...] = wide_ref[:, :DIM].T


_tc_narrow = pl.pallas_call(
    _tc_narrow_body,
    grid=(BATCH // _TC_ROWS,),
    in_specs=[pl.BlockSpec((_TC_ROWS, WDIM), lambda i: (i, 0))],
    out_specs=pl.BlockSpec((DIM, _TC_ROWS), lambda i: (0, i)),
    out_shape=jax.ShapeDtypeStruct((DIM, BATCH), jnp.float32),
)


def kernel(inputs, train_labels, embeddings):
    del train_labels
    table_t = embeddings.T
    tail_t = jnp.pad(embeddings[TAIL_START:], ((0, 128 - TAIL_COLS), (0, 0))).T
    wide = _sc_scan(inputs, table_t, tail_t)
    return _tc_narrow(wide).T
